# Initial kernel scaffold; baseline (speedup 1.0000x reference)
import jax
import jax.numpy as jnp
from jax.experimental import pallas as pl


def kernel(input, W_proj, means, log_vars, mix_logits):
    raise NotImplementedError("write your pallas kernel here")


# speedup vs baseline: 2.3551x; 2.3551x over previous
import math

import jax
import jax.numpy as jnp
from jax.experimental import pallas as pl
from jax.experimental.pallas import tpu as pltpu

MODEL_DIM = 1024
PROJ_DIM = 4
NUM_EXPERTS = 64
COMPONENTS = 4
TC = NUM_EXPERTS * COMPONENTS
S_TOTAL = 16384
BLOCK_S = 1024


def _gmm_kernel(x_ref, w_ref, nat_ref, grp_ref, post_ref, exp_ref, nll_ref):
    x = x_ref[...]
    w = w_ref[...]
    proj = jnp.dot(x, w, preferred_element_type=jnp.float32)

    bs = x.shape[0]
    acc = jnp.zeros((bs, TC), dtype=jnp.float32)
    for p in range(PROJ_DIM):
        d = proj[:, p:p + 1] - nat_ref[p:p + 1, :]
        acc = acc + d * d * nat_ref[PROJ_DIM + p:PROJ_DIM + p + 1, :]
    logits = nat_ref[2 * PROJ_DIM:2 * PROJ_DIM + 1, :] - 0.5 * acc

    m = jnp.max(logits, axis=-1, keepdims=True)
    e = jnp.exp(logits - m)
    z = jnp.sum(e, axis=-1, keepdims=True)
    inv_z = 1.0 / z
    post_ref[...] = e * inv_z

    gmax = None
    for k in range(COMPONENTS):
        acc2 = jnp.zeros((bs, NUM_EXPERTS), dtype=jnp.float32)
        for p in range(PROJ_DIM):
            row = k * PROJ_DIM + p
            d = proj[:, p:p + 1] - grp_ref[row:row + 1, :]
            acc2 = acc2 + d * d * grp_ref[16 + row:16 + row + 1, :]
        lk = grp_ref[32 + k:32 + k + 1, :] - 0.5 * acc2
        gmax = lk if gmax is None else jnp.maximum(gmax, lk)
    exp_ref[...] = jnp.exp(gmax - m) * inv_z

    s = jnp.sum(m[:, 0] + jnp.log(z[:, 0]))
    nll_ref[...] = jnp.broadcast_to(s, (1, 1, 128))


def kernel(input, W_proj, means, log_vars, mix_logits):
    S = input.shape[0]
    n_blocks = S // BLOCK_S

    mix_prob = jax.nn.softmax(jax.lax.stop_gradient(mix_logits))
    drop_mask = jax.random.bernoulli(jax.random.key(42), mix_prob)
    log_mix = jax.nn.log_softmax(mix_logits)
    vars_ = jnp.exp(log_vars)
    inv_v = 1.0 / (vars_ + 1e-06)
    log_det = jnp.sum(log_vars, axis=-1)
    bias = log_mix - 0.5 * (log_det + PROJ_DIM * math.log(2 * math.pi))
    bias = jnp.where(drop_mask, -1e30, bias)

    nat = jnp.zeros((16, TC), dtype=jnp.float32)
    nat = nat.at[0:PROJ_DIM, :].set(means.T)
    nat = nat.at[PROJ_DIM:2 * PROJ_DIM, :].set(inv_v.T)
    nat = nat.at[2 * PROJ_DIM, :].set(bias)

    means_g = means.reshape(NUM_EXPERTS, COMPONENTS, PROJ_DIM)
    inv_g = inv_v.reshape(NUM_EXPERTS, COMPONENTS, PROJ_DIM)
    bias_g = bias.reshape(NUM_EXPERTS, COMPONENTS)
    grp = jnp.zeros((40, NUM_EXPERTS), dtype=jnp.float32)
    grp = grp.at[0:16, :].set(
        means_g.transpose(1, 2, 0).reshape(16, NUM_EXPERTS))
    grp = grp.at[16:32, :].set(
        inv_g.transpose(1, 2, 0).reshape(16, NUM_EXPERTS))
    grp = grp.at[32:36, :].set(bias_g.T)

    post, expp, nll_parts = pl.pallas_call(
        _gmm_kernel,
        grid=(n_blocks,),
        in_specs=[
            pl.BlockSpec((BLOCK_S, MODEL_DIM), lambda i: (i, 0)),
            pl.BlockSpec((MODEL_DIM, PROJ_DIM), lambda i: (0, 0)),
            pl.BlockSpec((16, TC), lambda i: (0, 0)),
            pl.BlockSpec((40, NUM_EXPERTS), lambda i: (0, 0)),
        ],
        out_specs=[
            pl.BlockSpec((BLOCK_S, TC), lambda i: (i, 0)),
            pl.BlockSpec((BLOCK_S, NUM_EXPERTS), lambda i: (i, 0)),
            pl.BlockSpec((1, 1, 128), lambda i: (i, 0, 0)),
        ],
        out_shape=[
            jax.ShapeDtypeStruct((S, TC), jnp.float32),
            jax.ShapeDtypeStruct((S, NUM_EXPERTS), jnp.float32),
            jax.ShapeDtypeStruct((n_blocks, 1, 128), jnp.float32),
        ],
        compiler_params=pltpu.CompilerParams(
            dimension_semantics=("parallel",),
        ),
    )(input, W_proj, nat, grp)

    nll = -(jnp.sum(nll_parts[:, 0, 0]) / S)
    return (expp, post, nll)


# speedup vs baseline: 2.4056x; 1.0214x over previous
import math

import jax
import jax.numpy as jnp
from jax.experimental import pallas as pl
from jax.experimental.pallas import tpu as pltpu

MODEL_DIM = 1024
PROJ_DIM = 4
NUM_EXPERTS = 64
COMPONENTS = 4
TC = NUM_EXPERTS * COMPONENTS
BLOCK_S = 1024


def _gmm_kernel(x_ref, w_ref, a_ref, b_ref, bias_ref, post_ref, exp_ref,
                nll_ref):
    x = x_ref[...]
    proj = jnp.dot(x, w_ref[...], preferred_element_type=jnp.float32)
    psq = proj * proj
    lg = (jnp.dot(proj, a_ref[...], preferred_element_type=jnp.float32)
          + jnp.dot(psq, b_ref[...], preferred_element_type=jnp.float32)
          + bias_ref[...])

    logits = lg[:, :TC]
    m = jnp.max(logits, axis=-1, keepdims=True)
    e = jnp.exp(logits - m)
    z = jnp.sum(e, axis=-1, keepdims=True)
    inv_z = 1.0 / z
    post_ref[...] = e * inv_z

    g0 = jnp.maximum(lg[:, TC:TC + NUM_EXPERTS],
                     lg[:, TC + NUM_EXPERTS:TC + 2 * NUM_EXPERTS])
    g1 = jnp.maximum(lg[:, TC + 2 * NUM_EXPERTS:TC + 3 * NUM_EXPERTS],
                     lg[:, TC + 3 * NUM_EXPERTS:])
    gmax = jnp.maximum(g0, g1)
    exp_ref[...] = jnp.exp(gmax - m) * inv_z

    s = jnp.sum(m[:, 0] + jnp.log(z[:, 0]))
    nll_ref[...] = jnp.broadcast_to(s, (1, 1, 128))


def kernel(input, W_proj, means, log_vars, mix_logits):
    S = input.shape[0]
    n_blocks = S // BLOCK_S

    mix_prob = jax.nn.softmax(jax.lax.stop_gradient(mix_logits))
    drop_mask = jax.random.bernoulli(jax.random.key(42), mix_prob)
    log_mix = jax.nn.log_softmax(mix_logits)
    vars_ = jnp.exp(log_vars)
    inv_v = 1.0 / (vars_ + 1e-06)
    log_det = jnp.sum(log_vars, axis=-1)
    bias0 = log_mix - 0.5 * (log_det + PROJ_DIM * math.log(2 * math.pi)
                             + jnp.sum(means * means * inv_v, axis=-1))
    bias0 = jnp.where(drop_mask, -1e30, bias0)

    a0 = (means * inv_v).T
    b0 = (-0.5 * inv_v).T
    c = jnp.arange(TC)
    perm = (c % COMPONENTS) * NUM_EXPERTS + c // COMPONENTS
    inv_perm = jnp.argsort(perm)
    A = jnp.concatenate([a0, a0[:, inv_perm]], axis=1)
    B = jnp.concatenate([b0, b0[:, inv_perm]], axis=1)
    bias = jnp.concatenate([bias0, bias0[inv_perm]])[None, :]

    post, expp, nll_parts = pl.pallas_call(
        _gmm_kernel,
        grid=(n_blocks,),
        in_specs=[
            pl.BlockSpec((BLOCK_S, MODEL_DIM), lambda i: (i, 0)),
            pl.BlockSpec((MODEL_DIM, PROJ_DIM), lambda i: (0, 0)),
            pl.BlockSpec((PROJ_DIM, 2 * TC), lambda i: (0, 0)),
            pl.BlockSpec((PROJ_DIM, 2 * TC), lambda i: (0, 0)),
            pl.BlockSpec((1, 2 * TC), lambda i: (0, 0)),
        ],
        out_specs=[
            pl.BlockSpec((BLOCK_S, TC), lambda i: (i, 0)),
            pl.BlockSpec((BLOCK_S, NUM_EXPERTS), lambda i: (i, 0)),
            pl.BlockSpec((1, 1, 128), lambda i: (i, 0, 0)),
        ],
        out_shape=[
            jax.ShapeDtypeStruct((S, TC), jnp.float32),
            jax.ShapeDtypeStruct((S, NUM_EXPERTS), jnp.float32),
            jax.ShapeDtypeStruct((n_blocks, 1, 128), jnp.float32),
        ],
        compiler_params=pltpu.CompilerParams(
            dimension_semantics=("parallel",),
        ),
    )(input, W_proj, A, B, bias)

    nll = -(jnp.sum(nll_parts[:, 0, 0]) / S)
    return (expp, post, nll)
